# baseline (speedup 1.0000x reference)
import jax
import jax.numpy as jnp
from jax.experimental import pallas as pl
from jax.experimental.pallas import tpu as pltpu

_BM = 400
_BK = 2048
_NSTREAM = 6


def kernel(x, adj, W1, b1, W2):
    n, nfeat = x.shape
    nhid = W1.shape[1]
    nclass = W2.shape[1]
    bm = _BM
    bk = _BK
    nblk = n // bm
    nkblk = -(-n // bk)
    n_pad = nkblk * bk
    ncat = nhid + nclass

    def _pass_a_kernel(adj_ref, x_ref, w1_ref, b1_ref, w2_ref,
                       hid_ref, s2_ref, part_ref, cat_ref):
        t = pl.program_id(0)

        @pl.when(t == 0)
        def _():
            cat_ref[:, nhid:] = jnp.zeros((n, nclass), jnp.float32)
            cat_ref[:, :nhid] = jnp.dot(x_ref[...], w1_ref[...],
                                        preferred_element_type=jnp.float32)

        both = jnp.dot(adj_ref[...], cat_ref[...],
                       preferred_element_type=jnp.float32)
        h = jnp.maximum(both[:, :nhid] + b1_ref[...], 0.0)
        hid_ref[...] = h
        part_ref[...] = both[:, nhid:]
        s2_blk = jnp.dot(h, w2_ref[...], preferred_element_type=jnp.float32)
        cat_ref[pl.ds(t * bm, bm), nhid:] = s2_blk
        s2_ref[...] = s2_blk

    hid, s2, part = pl.pallas_call(
        _pass_a_kernel,
        grid=(nblk,),
        in_specs=[pl.BlockSpec((bm, n), lambda t: (t, 0)),
                  pl.BlockSpec((n, nfeat), lambda t: (0, 0)),
                  pl.BlockSpec((nfeat, nhid), lambda t: (0, 0)),
                  pl.BlockSpec((1, nhid), lambda t: (0, 0)),
                  pl.BlockSpec((nhid, nclass), lambda t: (0, 0))],
        out_specs=[pl.BlockSpec((bm, nhid), lambda t: (t, 0)),
                   pl.BlockSpec((bm, nclass), lambda t: (t, 0)),
                   pl.BlockSpec((bm, nclass), lambda t: (t, 0))],
        out_shape=[jax.ShapeDtypeStruct((n, nhid), jnp.float32),
                   jax.ShapeDtypeStruct((n, nclass), jnp.float32),
                   jax.ShapeDtypeStruct((n, nclass), jnp.float32)],
        scratch_shapes=[pltpu.VMEM((n, ncat), jnp.float32)],
        compiler_params=pltpu.CompilerParams(
            dimension_semantics=("arbitrary",),
            vmem_limit_bytes=63 * 1024 * 1024),
    )(adj, x, W1, b1.reshape(1, nhid), W2)

    s2p = jnp.pad(s2, ((0, n_pad - n), (0, 0)))

    ents = []
    for t in range(nblk):
        for k in range((t * bm) // bk, nkblk):
            ents.append((t, k, t * bm))
    while len(ents) % _NSTREAM:
        ents.append((0, 0, n))
    lanes = [ents[s::_NSTREAM] for s in range(_NSTREAM)]
    nsteps = len(lanes[0])
    idx = jnp.asarray(
        [row for lane in lanes for row in
         ([a[0] for a in lane], [a[1] for a in lane], [a[2] for a in lane])],
        dtype=jnp.int32)

    def _pass_b_kernel(idx_ref, *refs):
        a_refs = refs[:_NSTREAM]
        s2_ref, part_ref, out_ref, acc_ref = refs[_NSTREAM:]
        i = pl.program_id(0)

        @pl.when(i == 0)
        def _():
            acc_ref[...] = part_ref[...]

        for s in range(_NSTREAM):
            t = idx_ref[3 * s + 0, i]
            k = idx_ref[3 * s + 1, i]
            lo = idx_ref[3 * s + 2, i]
            col = k * bk + jax.lax.broadcasted_iota(jnp.int32, (1, bk), 1)
            a = jnp.where((col >= lo) & (col < n), a_refs[s][...], 0.0)
            acc_ref[pl.ds(t * bm, bm), :] += jnp.dot(
                a, s2_ref[pl.ds(k * bk, bk), :],
                preferred_element_type=jnp.float32)

        @pl.when(i == nsteps - 1)
        def _():
            out_ref[...] = acc_ref[...]

    adj_spec = [
        pl.BlockSpec(
            (bm, bk),
            (lambda s: lambda i, idx_ref:
             (idx_ref[3 * s, i], idx_ref[3 * s + 1, i]))(s))
        for s in range(_NSTREAM)
    ]

    out = pl.pallas_call(
        _pass_b_kernel,
        grid_spec=pltpu.PrefetchScalarGridSpec(
            num_scalar_prefetch=1,
            grid=(nsteps,),
            in_specs=adj_spec + [
                pl.BlockSpec((n_pad, nclass), lambda i, idx_ref: (0, 0)),
                pl.BlockSpec((n, nclass), lambda i, idx_ref: (0, 0)),
            ],
            out_specs=pl.BlockSpec((n, nclass), lambda i, idx_ref: (0, 0)),
            scratch_shapes=[pltpu.VMEM((n, nclass), jnp.float32)],
        ),
        out_shape=jax.ShapeDtypeStruct((n, nclass), jnp.float32),
        compiler_params=pltpu.CompilerParams(
            dimension_semantics=("arbitrary",),
            vmem_limit_bytes=63 * 1024 * 1024),
    )(idx, *([adj] * _NSTREAM), s2p, part)

    return (hid, out)


# speedup vs baseline: 1.0278x; 1.0278x over previous
import jax
import jax.numpy as jnp
from jax.experimental import pallas as pl
from jax.experimental.pallas import tpu as pltpu

_BM = 400
_BK = 2048
_NSTREAM = 4


def kernel(x, adj, W1, b1, W2):
    n, nfeat = x.shape
    nhid = W1.shape[1]
    nclass = W2.shape[1]
    bm = _BM
    bk = _BK
    nblk = n // bm
    nkblk = -(-n // bk)
    n_pad = nkblk * bk
    ncat = nhid + nclass

    def _pass_a_kernel(adj_ref, x_ref, w1_ref, b1_ref, w2_ref,
                       hid_ref, s2_ref, part_ref, cat_ref):
        t = pl.program_id(0)

        @pl.when(t == 0)
        def _():
            cat_ref[:, nhid:] = jnp.zeros((n, nclass), jnp.float32)
            cat_ref[:, :nhid] = jnp.dot(x_ref[...], w1_ref[...],
                                        preferred_element_type=jnp.float32)

        both = jnp.dot(adj_ref[...], cat_ref[...],
                       preferred_element_type=jnp.float32)
        h = jnp.maximum(both[:, :nhid] + b1_ref[...], 0.0)
        hid_ref[...] = h
        part_ref[...] = both[:, nhid:]
        s2_blk = jnp.dot(h, w2_ref[...], preferred_element_type=jnp.float32)
        cat_ref[pl.ds(t * bm, bm), nhid:] = s2_blk
        s2_ref[...] = s2_blk

    hid, s2, part = pl.pallas_call(
        _pass_a_kernel,
        grid=(nblk,),
        in_specs=[pl.BlockSpec((bm, n), lambda t: (t, 0)),
                  pl.BlockSpec((n, nfeat), lambda t: (0, 0)),
                  pl.BlockSpec((nfeat, nhid), lambda t: (0, 0)),
                  pl.BlockSpec((1, nhid), lambda t: (0, 0)),
                  pl.BlockSpec((nhid, nclass), lambda t: (0, 0))],
        out_specs=[pl.BlockSpec((bm, nhid), lambda t: (t, 0)),
                   pl.BlockSpec((bm, nclass), lambda t: (t, 0)),
                   pl.BlockSpec((bm, nclass), lambda t: (t, 0))],
        out_shape=[jax.ShapeDtypeStruct((n, nhid), jnp.float32),
                   jax.ShapeDtypeStruct((n, nclass), jnp.float32),
                   jax.ShapeDtypeStruct((n, nclass), jnp.float32)],
        scratch_shapes=[pltpu.VMEM((n, ncat), jnp.float32)],
        compiler_params=pltpu.CompilerParams(
            dimension_semantics=("arbitrary",),
            vmem_limit_bytes=63 * 1024 * 1024),
    )(adj, x, W1, b1.reshape(1, nhid), W2)

    s2p = jnp.pad(s2, ((0, n_pad - n), (0, 0)))

    ents = []
    for t in range(nblk):
        for k in range((t * bm) // bk, nkblk):
            ents.append((t, k, t * bm))
    while len(ents) % _NSTREAM:
        prev = ents[-_NSTREAM]
        ents.append((prev[0], prev[1], n))
    lanes = [ents[s::_NSTREAM] for s in range(_NSTREAM)]
    nsteps = len(lanes[0])
    idx = jnp.asarray(
        [row for lane in lanes for row in
         ([a[0] for a in lane], [a[1] for a in lane], [a[2] for a in lane])],
        dtype=jnp.int32)

    def _pass_b_kernel(idx_ref, *refs):
        a_refs = refs[:_NSTREAM]
        s2_ref, part_ref, out_ref, acc_ref = refs[_NSTREAM:]
        i = pl.program_id(0)

        @pl.when(i == 0)
        def _():
            acc_ref[...] = part_ref[...]

        for s in range(_NSTREAM):
            t = idx_ref[3 * s + 0, i]
            k = idx_ref[3 * s + 1, i]
            lo = idx_ref[3 * s + 2, i]
            col = k * bk + jax.lax.broadcasted_iota(jnp.int32, (1, bk), 1)
            a = jnp.where((col >= lo) & (col < n), a_refs[s][...], 0.0)
            acc_ref[pl.ds(t * bm, bm), :] += jnp.dot(
                a, s2_ref[pl.ds(k * bk, bk), :],
                preferred_element_type=jnp.float32)

        @pl.when(i == nsteps - 1)
        def _():
            out_ref[...] = acc_ref[...]

    adj_spec = [
        pl.BlockSpec(
            (bm, bk),
            (lambda s: lambda i, idx_ref:
             (idx_ref[3 * s, i], idx_ref[3 * s + 1, i]))(s))
        for s in range(_NSTREAM)
    ]

    out = pl.pallas_call(
        _pass_b_kernel,
        grid_spec=pltpu.PrefetchScalarGridSpec(
            num_scalar_prefetch=1,
            grid=(nsteps,),
            in_specs=adj_spec + [
                pl.BlockSpec((n_pad, nclass), lambda i, idx_ref: (0, 0)),
                pl.BlockSpec((n, nclass), lambda i, idx_ref: (0, 0)),
            ],
            out_specs=pl.BlockSpec((n, nclass), lambda i, idx_ref: (0, 0)),
            scratch_shapes=[pltpu.VMEM((n, nclass), jnp.float32)],
        ),
        out_shape=jax.ShapeDtypeStruct((n, nclass), jnp.float32),
        compiler_params=pltpu.CompilerParams(
            dimension_semantics=("arbitrary",),
            vmem_limit_bytes=63 * 1024 * 1024),
    )(idx, *([adj] * _NSTREAM), s2p, part)

    return (hid, out)
